# baseline (speedup 1.0000x reference)
import functools

import jax
import jax.numpy as jnp
from jax import lax
from jax.experimental import pallas as pl
from jax.experimental.pallas import tpu as pltpu
from jax.experimental.pallas import tpu_sc as plsc

N = 100000
B = 512
D = 128
H = 64


TN = 2000
GRID = N // TN


def _mlp_body(x_ref, w1_ref, b1_ref, w2_ref, b2_ref, out_ref):
    x = x_ref[...]
    h = jnp.dot(x, w1_ref[...], preferred_element_type=jnp.float32) + b1_ref[...]
    h = h * jax.nn.sigmoid(h)
    q = jnp.dot(h, w2_ref[...], preferred_element_type=jnp.float32) + b2_ref[...]
    out_ref[...] = q


def _mlp(scaler, W1, b1, W2, b2):
    return pl.pallas_call(
        _mlp_body,
        grid=(GRID,),
        in_specs=[
            pl.BlockSpec((TN, D), lambda i: (i, 0)),
            pl.BlockSpec((D, H), lambda i: (0, 0)),
            pl.BlockSpec((1, H), lambda i: (0, 0)),
            pl.BlockSpec((H, 1), lambda i: (0, 0)),
            pl.BlockSpec((1, 1), lambda i: (0, 0)),
        ],
        out_specs=pl.BlockSpec((TN, 1), lambda i: (i, 0)),
        out_shape=jax.ShapeDtypeStruct((N, 1), jnp.float32),
        compiler_params=pltpu.CompilerParams(
            dimension_semantics=("arbitrary",),
        ),
    )(scaler, W1, b1.reshape(1, H), W2, b2.reshape(1, 1))


NWORK = 16
STEPS = 391
CHUNK = STEPS * 16
N_PAD = NWORK * CHUNK
BROWS = B // 16

_sc_mesh = plsc.VectorSubcoreMesh(
    core_axis_name="c", subcore_axis_name="s", num_cores=1)


@functools.partial(
    pl.kernel,
    mesh=_sc_mesh,
    out_type=jax.ShapeDtypeStruct((B,), jnp.float32),
    scratch_types=[
        pltpu.VMEM((CHUNK,), jnp.float32),
        pltpu.VMEM((3 * CHUNK,), jnp.float32),
        pltpu.VMEM((CHUNK,), jnp.int32),
        pltpu.VMEM((3 * B,), jnp.float32),
        pltpu.VMEM((16, B), jnp.float32),
        pltpu.VMEM((B,), jnp.float32),
        pltpu.VMEM_SHARED((NWORK, B), jnp.float32),
    ],
    compiler_params=pltpu.CompilerParams(needs_layout_passes=False),
)
def _sc_segsum(q_hbm, pos_hbm, bi_hbm, mc_hbm, out_hbm,
               qv, posv, biv, mcv, lacc, acc, shared):
    wid = lax.axis_index("s")
    base = wid * CHUNK
    pltpu.sync_copy(q_hbm.at[pl.ds(base, CHUNK)], qv)
    pltpu.sync_copy(pos_hbm.at[pl.ds(3 * base, 3 * CHUNK)], posv)
    pltpu.sync_copy(bi_hbm.at[pl.ds(base, CHUNK)], biv)
    pltpu.sync_copy(mc_hbm, mcv)

    zeros16 = jnp.zeros((16,), jnp.float32)
    lanes = lax.iota(jnp.int32, 16)

    def zbody(j, carry):
        off = j * 16
        for l in range(16):
            lacc[l, pl.ds(off, 16)] = zeros16
        return carry

    lax.fori_loop(0, BROWS, zbody, 0)

    lanes3 = lanes * 3

    def body(j, carry):
        off = j * 16
        idx = biv[pl.ds(off, 16)]
        q = qv[pl.ds(off, 16)]
        p3 = off * 3 + lanes3
        x = plsc.load_gather(posv, [p3])
        y = plsc.load_gather(posv, [p3 + 1])
        z = plsc.load_gather(posv, [p3 + 2])
        m3 = idx * 3
        dx = x - plsc.load_gather(mcv, [m3])
        dy = y - plsc.load_gather(mcv, [m3 + 1])
        dz = z - plsc.load_gather(mcv, [m3 + 2])
        v = q * (dx * dx + dy * dy + dz * dz)
        plsc.addupdate_scatter(lacc, [lanes, idx], v)
        return carry

    lax.fori_loop(0, STEPS, body, 0)

    def fbody(g, carry):
        off = g * 16
        s = lacc[0, pl.ds(off, 16)]
        for l in range(1, 16):
            s = s + lacc[l, pl.ds(off, 16)]
        acc[pl.ds(off, 16)] = s
        return carry

    lax.fori_loop(0, BROWS, fbody, 0)

    pltpu.sync_copy(acc, shared.at[wid])
    plsc.subcore_barrier()

    @pl.when(wid == 0)
    def _writeout():
        pltpu.sync_copy(shared, lacc)

        def gbody(g, carry):
            off = g * 16
            s = lacc[0, pl.ds(off, 16)]
            for l in range(1, NWORK):
                s = s + lacc[l, pl.ds(off, 16)]
            acc[pl.ds(off, 16)] = s
            return carry

        lax.fori_loop(0, BROWS, gbody, 0)
        pltpu.sync_copy(acc, out_hbm)


def kernel(pos, mass_center, scaler, vector, batch_index, W1, b1, W2, b2):
    del vector
    q = _mlp(scaler, W1, b1, W2, b2)[:, 0]

    pad = N_PAD - N
    qp = jnp.concatenate([q, jnp.zeros((pad,), jnp.float32)])
    posf = jnp.concatenate([pos.reshape(3 * N), jnp.zeros((3 * pad,), jnp.float32)])
    bi = jnp.concatenate([batch_index.astype(jnp.int32),
                          jnp.zeros((pad,), jnp.int32)])
    mcf = mass_center.reshape(3 * B)

    return _sc_segsum(qp, posf, bi, mcf).reshape(B, 1)


# speedup vs baseline: 1.6332x; 1.6332x over previous
import functools

import jax
import jax.numpy as jnp
from jax import lax
from jax.experimental import pallas as pl
from jax.experimental.pallas import tpu as pltpu
from jax.experimental.pallas import tpu_sc as plsc

N = 100000
B = 512
D = 128
H = 64


TN = 4000
GRID = N // TN


def _mlp_body(x_ref, w1_ref, b1_ref, w2_ref, b2_ref, out_ref):
    x = x_ref[...]
    h = jnp.dot(x, w1_ref[...], preferred_element_type=jnp.float32) + b1_ref[...]
    h = h * jax.nn.sigmoid(h)
    q = jnp.dot(h, w2_ref[...], preferred_element_type=jnp.float32) + b2_ref[...]
    out_ref[...] = q


def _mlp(scaler, W1, b1, W2, b2):
    return pl.pallas_call(
        _mlp_body,
        grid=(GRID,),
        in_specs=[
            pl.BlockSpec((TN, D), lambda i: (i, 0)),
            pl.BlockSpec((D, H), lambda i: (0, 0)),
            pl.BlockSpec((1, H), lambda i: (0, 0)),
            pl.BlockSpec((H, 1), lambda i: (0, 0)),
            pl.BlockSpec((1, 1), lambda i: (0, 0)),
        ],
        out_specs=pl.BlockSpec((TN, 1), lambda i: (i, 0)),
        out_shape=jax.ShapeDtypeStruct((N, 1), jnp.float32),
        compiler_params=pltpu.CompilerParams(
            dimension_semantics=("parallel",),
        ),
    )(scaler, W1, b1.reshape(1, H), W2, b2.reshape(1, 1))


NWORK = 16
STEPS = 391
CHUNK = STEPS * 16
N_PAD = NWORK * CHUNK
BROWS = B // 16

_sc_mesh = plsc.VectorSubcoreMesh(
    core_axis_name="c", subcore_axis_name="s", num_cores=1)


@functools.partial(
    pl.kernel,
    mesh=_sc_mesh,
    out_type=jax.ShapeDtypeStruct((B,), jnp.float32),
    scratch_types=[
        pltpu.VMEM((CHUNK,), jnp.float32),
        pltpu.VMEM((CHUNK,), jnp.float32),
        pltpu.VMEM((CHUNK,), jnp.float32),
        pltpu.VMEM((CHUNK,), jnp.float32),
        pltpu.VMEM((CHUNK,), jnp.int32),
        pltpu.VMEM((B,), jnp.float32),
        pltpu.VMEM((B,), jnp.float32),
        pltpu.VMEM((B,), jnp.float32),
        pltpu.VMEM((16, B), jnp.float32),
        pltpu.VMEM((B,), jnp.float32),
        pltpu.VMEM_SHARED((NWORK, B), jnp.float32),
    ],
    compiler_params=pltpu.CompilerParams(needs_layout_passes=False),
)
def _sc_segsum(q_hbm, px_hbm, py_hbm, pz_hbm, bi_hbm,
               mcx_hbm, mcy_hbm, mcz_hbm, out_hbm,
               qv, xv, yv, zv, biv, mcx, mcy, mcz, lacc, acc, shared):
    wid = lax.axis_index("s")
    base = wid * CHUNK
    pltpu.sync_copy(q_hbm.at[pl.ds(base, CHUNK)], qv)
    pltpu.sync_copy(px_hbm.at[pl.ds(base, CHUNK)], xv)
    pltpu.sync_copy(py_hbm.at[pl.ds(base, CHUNK)], yv)
    pltpu.sync_copy(pz_hbm.at[pl.ds(base, CHUNK)], zv)
    pltpu.sync_copy(bi_hbm.at[pl.ds(base, CHUNK)], biv)
    pltpu.sync_copy(mcx_hbm, mcx)
    pltpu.sync_copy(mcy_hbm, mcy)
    pltpu.sync_copy(mcz_hbm, mcz)

    zeros16 = jnp.zeros((16,), jnp.float32)
    lanes = lax.iota(jnp.int32, 16)

    def zbody(j, carry):
        off = j * 16
        for l in range(16):
            lacc[l, pl.ds(off, 16)] = zeros16
        return carry

    lax.fori_loop(0, BROWS, zbody, 0)

    def body(j, carry):
        off = j * 16
        idx = biv[pl.ds(off, 16)]
        q = qv[pl.ds(off, 16)]
        x = xv[pl.ds(off, 16)]
        y = yv[pl.ds(off, 16)]
        z = zv[pl.ds(off, 16)]
        dx = x - plsc.load_gather(mcx, [idx])
        dy = y - plsc.load_gather(mcy, [idx])
        dz = z - plsc.load_gather(mcz, [idx])
        v = q * (dx * dx + dy * dy + dz * dz)
        plsc.addupdate_scatter(lacc, [lanes, idx], v)
        return carry

    lax.fori_loop(0, STEPS, body, 0)

    def fbody(g, carry):
        off = g * 16
        s = lacc[0, pl.ds(off, 16)]
        for l in range(1, 16):
            s = s + lacc[l, pl.ds(off, 16)]
        acc[pl.ds(off, 16)] = s
        return carry

    lax.fori_loop(0, BROWS, fbody, 0)

    pltpu.sync_copy(acc, shared.at[wid])
    plsc.subcore_barrier()

    @pl.when(wid == 0)
    def _writeout():
        pltpu.sync_copy(shared, lacc)

        def gbody(g, carry):
            off = g * 16
            s = lacc[0, pl.ds(off, 16)]
            for l in range(1, NWORK):
                s = s + lacc[l, pl.ds(off, 16)]
            acc[pl.ds(off, 16)] = s
            return carry

        lax.fori_loop(0, BROWS, gbody, 0)
        pltpu.sync_copy(acc, out_hbm)


def kernel(pos, mass_center, scaler, vector, batch_index, W1, b1, W2, b2):
    del vector
    q = _mlp(scaler, W1, b1, W2, b2)[:, 0]

    pad = N_PAD - N
    qp = jnp.concatenate([q, jnp.zeros((pad,), jnp.float32)])
    posT = pos.T
    px = jnp.concatenate([posT[0], jnp.zeros((pad,), jnp.float32)])
    py = jnp.concatenate([posT[1], jnp.zeros((pad,), jnp.float32)])
    pz = jnp.concatenate([posT[2], jnp.zeros((pad,), jnp.float32)])
    bi = jnp.concatenate([batch_index.astype(jnp.int32),
                          jnp.zeros((pad,), jnp.int32)])
    mcT = mass_center.T

    return _sc_segsum(qp, px, py, pz, bi, mcT[0], mcT[1], mcT[2]).reshape(B, 1)


# speedup vs baseline: 1.7000x; 1.0409x over previous
import functools

import jax
import jax.numpy as jnp
from jax import lax
from jax.experimental import pallas as pl
from jax.experimental.pallas import tpu as pltpu
from jax.experimental.pallas import tpu_sc as plsc

N = 100000
B = 512
D = 128
H = 64


TN = 4000
GRID = N // TN


def _mlp_body(x_ref, w1_ref, b1_ref, w2_ref, b2_ref, out_ref):
    x = x_ref[...]
    h = jnp.dot(x, w1_ref[...], preferred_element_type=jnp.float32) + b1_ref[...]
    h = h * jax.nn.sigmoid(h)
    q = jnp.dot(h, w2_ref[...], preferred_element_type=jnp.float32) + b2_ref[...]
    out_ref[...] = q


def _mlp(scaler, W1, b1, W2, b2):
    return pl.pallas_call(
        _mlp_body,
        grid=(GRID,),
        in_specs=[
            pl.BlockSpec((TN, D), lambda i: (i, 0)),
            pl.BlockSpec((D, H), lambda i: (0, 0)),
            pl.BlockSpec((1, H), lambda i: (0, 0)),
            pl.BlockSpec((H, 1), lambda i: (0, 0)),
            pl.BlockSpec((1, 1), lambda i: (0, 0)),
        ],
        out_specs=pl.BlockSpec((TN, 1), lambda i: (i, 0)),
        out_shape=jax.ShapeDtypeStruct((N, 1), jnp.float32),
        compiler_params=pltpu.CompilerParams(
            dimension_semantics=("parallel",),
        ),
    )(scaler, W1, b1.reshape(1, H), W2, b2.reshape(1, 1))


NWORK = 16
STEPS = 391
CHUNK = STEPS * 16
N_PAD = NWORK * CHUNK
BROWS = B // 16

_sc_mesh = plsc.VectorSubcoreMesh(
    core_axis_name="c", subcore_axis_name="s", num_cores=1)


@functools.partial(
    pl.kernel,
    mesh=_sc_mesh,
    out_type=jax.ShapeDtypeStruct((B,), jnp.float32),
    scratch_types=[
        pltpu.VMEM((CHUNK,), jnp.float32),
        pltpu.VMEM((CHUNK,), jnp.float32),
        pltpu.VMEM((CHUNK,), jnp.float32),
        pltpu.VMEM((CHUNK,), jnp.float32),
        pltpu.VMEM((CHUNK,), jnp.int32),
        pltpu.VMEM((B,), jnp.float32),
        pltpu.VMEM((B,), jnp.float32),
        pltpu.VMEM((B,), jnp.float32),
        pltpu.VMEM((16, B + 1), jnp.float32),
        pltpu.VMEM((B,), jnp.float32),
        pltpu.VMEM((NWORK, B), jnp.float32),
        pltpu.VMEM_SHARED((NWORK, B), jnp.float32),
        pltpu.SemaphoreType.DMA,
    ],
    compiler_params=pltpu.CompilerParams(needs_layout_passes=False),
)
def _sc_segsum(q_hbm, px_hbm, py_hbm, pz_hbm, bi_hbm,
               mcx_hbm, mcy_hbm, mcz_hbm, out_hbm,
               qv, xv, yv, zv, biv, mcx, mcy, mcz, lacc, acc, fold, shared,
               dsem):
    wid = lax.axis_index("s")
    base = wid * CHUNK
    copies = [
        pltpu.async_copy(q_hbm.at[pl.ds(base, CHUNK)], qv, dsem),
        pltpu.async_copy(px_hbm.at[pl.ds(base, CHUNK)], xv, dsem),
        pltpu.async_copy(py_hbm.at[pl.ds(base, CHUNK)], yv, dsem),
        pltpu.async_copy(pz_hbm.at[pl.ds(base, CHUNK)], zv, dsem),
        pltpu.async_copy(bi_hbm.at[pl.ds(base, CHUNK)], biv, dsem),
        pltpu.async_copy(mcx_hbm, mcx, dsem),
        pltpu.async_copy(mcy_hbm, mcy, dsem),
        pltpu.async_copy(mcz_hbm, mcz, dsem),
    ]

    zeros16 = jnp.zeros((16,), jnp.float32)
    lanes = lax.iota(jnp.int32, 16)

    def zbody(j, carry):
        off = j * 16
        for l in range(16):
            lacc[l, pl.ds(off, 16)] = zeros16
        return carry

    lax.fori_loop(0, BROWS, zbody, 0)
    for c in copies:
        c.wait()

    def body(j, carry):
        off = j * 16
        idx = biv[pl.ds(off, 16)]
        q = qv[pl.ds(off, 16)]
        x = xv[pl.ds(off, 16)]
        y = yv[pl.ds(off, 16)]
        z = zv[pl.ds(off, 16)]
        dx = x - plsc.load_gather(mcx, [idx])
        dy = y - plsc.load_gather(mcy, [idx])
        dz = z - plsc.load_gather(mcz, [idx])
        v = q * (dx * dx + dy * dy + dz * dz)
        plsc.addupdate_scatter(lacc, [lanes, idx], v)
        return carry

    lax.fori_loop(0, STEPS, body, 0)

    def fbody(g, carry):
        off = g * 16
        s = lacc[0, pl.ds(off, 16)]
        for l in range(1, 16):
            s = s + lacc[l, pl.ds(off, 16)]
        acc[pl.ds(off, 16)] = s
        return carry

    lax.fori_loop(0, BROWS, fbody, 0)

    pltpu.sync_copy(acc, shared.at[wid])
    plsc.subcore_barrier()

    @pl.when(wid == 0)
    def _writeout():
        pltpu.sync_copy(shared, fold)

        def gbody(g, carry):
            off = g * 16
            s = fold[0, pl.ds(off, 16)]
            for l in range(1, NWORK):
                s = s + fold[l, pl.ds(off, 16)]
            acc[pl.ds(off, 16)] = s
            return carry

        lax.fori_loop(0, BROWS, gbody, 0)
        pltpu.sync_copy(acc, out_hbm)


def kernel(pos, mass_center, scaler, vector, batch_index, W1, b1, W2, b2):
    del vector
    q = _mlp(scaler, W1, b1, W2, b2)[:, 0]

    pad = N_PAD - N
    qp = jnp.concatenate([q, jnp.zeros((pad,), jnp.float32)])
    posT = pos.T
    px = jnp.concatenate([posT[0], jnp.zeros((pad,), jnp.float32)])
    py = jnp.concatenate([posT[1], jnp.zeros((pad,), jnp.float32)])
    pz = jnp.concatenate([posT[2], jnp.zeros((pad,), jnp.float32)])
    bi = jnp.concatenate([batch_index.astype(jnp.int32),
                          jnp.zeros((pad,), jnp.int32)])
    mcT = mass_center.T

    return _sc_segsum(qp, px, py, pz, bi, mcT[0], mcT[1], mcT[2]).reshape(B, 1)


# speedup vs baseline: 1.7008x; 1.0005x over previous
import functools

import jax
import jax.numpy as jnp
from jax import lax
from jax.experimental import pallas as pl
from jax.experimental.pallas import tpu as pltpu
from jax.experimental.pallas import tpu_sc as plsc

N = 100000
B = 512
D = 128
H = 64


TN = 4000
GRID = N // TN


def _mlp_body(x_ref, w1_ref, b1_ref, w2_ref, b2_ref, out_ref):
    x = x_ref[...]
    h = jnp.dot(x, w1_ref[...], preferred_element_type=jnp.float32) + b1_ref[...]
    h = h * (0.5 * jnp.tanh(0.5 * h) + 0.5)
    q = jnp.dot(h, w2_ref[...], preferred_element_type=jnp.float32) + b2_ref[...]
    out_ref[...] = q


def _mlp(scaler, W1, b1, W2, b2):
    return pl.pallas_call(
        _mlp_body,
        grid=(GRID,),
        in_specs=[
            pl.BlockSpec((TN, D), lambda i: (i, 0)),
            pl.BlockSpec((D, H), lambda i: (0, 0)),
            pl.BlockSpec((1, H), lambda i: (0, 0)),
            pl.BlockSpec((H, 1), lambda i: (0, 0)),
            pl.BlockSpec((1, 1), lambda i: (0, 0)),
        ],
        out_specs=pl.BlockSpec((TN, 1), lambda i: (i, 0)),
        out_shape=jax.ShapeDtypeStruct((N, 1), jnp.float32),
        compiler_params=pltpu.CompilerParams(
            dimension_semantics=("parallel",),
        ),
    )(scaler, W1, b1.reshape(1, H), W2, b2.reshape(1, 1))


NWORK = 16
STEPS = 391
CHUNK = STEPS * 16
N_PAD = NWORK * CHUNK
BROWS = B // 16

_sc_mesh = plsc.VectorSubcoreMesh(
    core_axis_name="c", subcore_axis_name="s", num_cores=1)


@functools.partial(
    pl.kernel,
    mesh=_sc_mesh,
    out_type=jax.ShapeDtypeStruct((B,), jnp.float32),
    scratch_types=[
        pltpu.VMEM((CHUNK,), jnp.float32),
        pltpu.VMEM((CHUNK,), jnp.float32),
        pltpu.VMEM((CHUNK,), jnp.float32),
        pltpu.VMEM((CHUNK,), jnp.float32),
        pltpu.VMEM((CHUNK,), jnp.int32),
        pltpu.VMEM((B,), jnp.float32),
        pltpu.VMEM((B,), jnp.float32),
        pltpu.VMEM((B,), jnp.float32),
        pltpu.VMEM((16, B + 1), jnp.float32),
        pltpu.VMEM((B,), jnp.float32),
        pltpu.VMEM((NWORK, B), jnp.float32),
        pltpu.VMEM_SHARED((NWORK, B), jnp.float32),
        pltpu.SemaphoreType.DMA,
    ],
    compiler_params=pltpu.CompilerParams(needs_layout_passes=False),
)
def _sc_segsum(q_hbm, px_hbm, py_hbm, pz_hbm, bi_hbm,
               mcx_hbm, mcy_hbm, mcz_hbm, out_hbm,
               qv, xv, yv, zv, biv, mcx, mcy, mcz, lacc, acc, fold, shared,
               dsem):
    wid = lax.axis_index("s")
    base = wid * CHUNK
    copies = [
        pltpu.async_copy(q_hbm.at[pl.ds(base, CHUNK)], qv, dsem),
        pltpu.async_copy(px_hbm.at[pl.ds(base, CHUNK)], xv, dsem),
        pltpu.async_copy(py_hbm.at[pl.ds(base, CHUNK)], yv, dsem),
        pltpu.async_copy(pz_hbm.at[pl.ds(base, CHUNK)], zv, dsem),
        pltpu.async_copy(bi_hbm.at[pl.ds(base, CHUNK)], biv, dsem),
        pltpu.async_copy(mcx_hbm, mcx, dsem),
        pltpu.async_copy(mcy_hbm, mcy, dsem),
        pltpu.async_copy(mcz_hbm, mcz, dsem),
    ]

    zeros16 = jnp.zeros((16,), jnp.float32)
    lanes = lax.iota(jnp.int32, 16)

    def zbody(j, carry):
        off = j * 16
        for l in range(16):
            lacc[l, pl.ds(off, 16)] = zeros16
        return carry

    lax.fori_loop(0, BROWS, zbody, 0)
    for c in copies:
        c.wait()

    def body(j, carry):
        off = j * 16
        idx = biv[pl.ds(off, 16)]
        q = qv[pl.ds(off, 16)]
        x = xv[pl.ds(off, 16)]
        y = yv[pl.ds(off, 16)]
        z = zv[pl.ds(off, 16)]
        dx = x - plsc.load_gather(mcx, [idx])
        dy = y - plsc.load_gather(mcy, [idx])
        dz = z - plsc.load_gather(mcz, [idx])
        v = q * (dx * dx + dy * dy + dz * dz)
        plsc.addupdate_scatter(lacc, [lanes, idx], v)
        return carry

    lax.fori_loop(0, STEPS, body, 0)

    def fbody(g, carry):
        off = g * 16
        s = lacc[0, pl.ds(off, 16)]
        for l in range(1, 16):
            s = s + lacc[l, pl.ds(off, 16)]
        acc[pl.ds(off, 16)] = s
        return carry

    lax.fori_loop(0, BROWS, fbody, 0)

    pltpu.sync_copy(acc, shared.at[wid])
    plsc.subcore_barrier()

    @pl.when(wid == 0)
    def _writeout():
        pltpu.sync_copy(shared, fold)

        def gbody(g, carry):
            off = g * 16
            s = fold[0, pl.ds(off, 16)]
            for l in range(1, NWORK):
                s = s + fold[l, pl.ds(off, 16)]
            acc[pl.ds(off, 16)] = s
            return carry

        lax.fori_loop(0, BROWS, gbody, 0)
        pltpu.sync_copy(acc, out_hbm)


def kernel(pos, mass_center, scaler, vector, batch_index, W1, b1, W2, b2):
    del vector
    q = _mlp(scaler, W1, b1, W2, b2)[:, 0]

    pad = N_PAD - N
    qp = jnp.concatenate([q, jnp.zeros((pad,), jnp.float32)])
    posT = pos.T
    px = jnp.concatenate([posT[0], jnp.zeros((pad,), jnp.float32)])
    py = jnp.concatenate([posT[1], jnp.zeros((pad,), jnp.float32)])
    pz = jnp.concatenate([posT[2], jnp.zeros((pad,), jnp.float32)])
    bi = jnp.concatenate([batch_index.astype(jnp.int32),
                          jnp.zeros((pad,), jnp.int32)])
    mcT = mass_center.T

    return _sc_segsum(qp, px, py, pz, bi, mcT[0], mcT[1], mcT[2]).reshape(B, 1)


# speedup vs baseline: 1.8481x; 1.0866x over previous
import functools

import jax
import jax.numpy as jnp
from jax import lax
from jax.experimental import pallas as pl
from jax.experimental.pallas import tpu as pltpu
from jax.experimental.pallas import tpu_sc as plsc

N = 100000
B = 512
D = 128
H = 64


TN = 10000
GRID = N // TN


def _mlp_body(x_ref, w1_ref, b1_ref, w2_ref, b2_ref, out_ref):
    x = x_ref[...]
    h = jnp.dot(x, w1_ref[...], preferred_element_type=jnp.float32) + b1_ref[...]
    h = h * jax.nn.sigmoid(h)
    q = jnp.dot(h, w2_ref[...], preferred_element_type=jnp.float32) + b2_ref[...]
    out_ref[...] = q


def _mlp(scaler, W1, b1, W2, b2):
    return pl.pallas_call(
        _mlp_body,
        grid=(GRID,),
        in_specs=[
            pl.BlockSpec((TN, D), lambda i: (i, 0)),
            pl.BlockSpec((D, H), lambda i: (0, 0)),
            pl.BlockSpec((1, H), lambda i: (0, 0)),
            pl.BlockSpec((H, 1), lambda i: (0, 0)),
            pl.BlockSpec((1, 1), lambda i: (0, 0)),
        ],
        out_specs=pl.BlockSpec((TN, 1), lambda i: (i, 0)),
        out_shape=jax.ShapeDtypeStruct((N, 1), jnp.float32),
        compiler_params=pltpu.CompilerParams(
            dimension_semantics=("parallel",),
        ),
    )(scaler, W1, b1.reshape(1, H), W2, b2.reshape(1, 1))


NWORK = 16
STEPS = 392
CHUNK = STEPS * 16
N_PAD = NWORK * CHUNK
BROWS = B // 16
UNROLL = 4

_sc_mesh = plsc.VectorSubcoreMesh(
    core_axis_name="c", subcore_axis_name="s", num_cores=1)


@functools.partial(
    pl.kernel,
    mesh=_sc_mesh,
    out_type=jax.ShapeDtypeStruct((B,), jnp.float32),
    scratch_types=[
        pltpu.VMEM((CHUNK,), jnp.float32),
        pltpu.VMEM((CHUNK,), jnp.float32),
        pltpu.VMEM((CHUNK,), jnp.float32),
        pltpu.VMEM((CHUNK,), jnp.float32),
        pltpu.VMEM((CHUNK,), jnp.int32),
        pltpu.VMEM((B,), jnp.float32),
        pltpu.VMEM((B,), jnp.float32),
        pltpu.VMEM((B,), jnp.float32),
        pltpu.VMEM((16, B + 1), jnp.float32),
        pltpu.VMEM((B,), jnp.float32),
        pltpu.VMEM((NWORK, B), jnp.float32),
        pltpu.VMEM_SHARED((NWORK, B), jnp.float32),
        pltpu.SemaphoreType.DMA,
    ],
    compiler_params=pltpu.CompilerParams(needs_layout_passes=False),
)
def _sc_segsum(q_hbm, px_hbm, py_hbm, pz_hbm, bi_hbm,
               mcx_hbm, mcy_hbm, mcz_hbm, out_hbm,
               qv, xv, yv, zv, biv, mcx, mcy, mcz, lacc, acc, fold, shared,
               dsem):
    wid = lax.axis_index("s")
    base = wid * CHUNK
    copies = [
        pltpu.async_copy(q_hbm.at[pl.ds(base, CHUNK)], qv, dsem),
        pltpu.async_copy(px_hbm.at[pl.ds(base, CHUNK)], xv, dsem),
        pltpu.async_copy(py_hbm.at[pl.ds(base, CHUNK)], yv, dsem),
        pltpu.async_copy(pz_hbm.at[pl.ds(base, CHUNK)], zv, dsem),
        pltpu.async_copy(bi_hbm.at[pl.ds(base, CHUNK)], biv, dsem),
        pltpu.async_copy(mcx_hbm, mcx, dsem),
        pltpu.async_copy(mcy_hbm, mcy, dsem),
        pltpu.async_copy(mcz_hbm, mcz, dsem),
    ]

    zeros16 = jnp.zeros((16,), jnp.float32)
    lanes = lax.iota(jnp.int32, 16)

    def zbody(j, carry):
        off = j * 16
        for l in range(16):
            lacc[l, pl.ds(off, 16)] = zeros16
        return carry

    lax.fori_loop(0, BROWS, zbody, 0)
    for c in copies:
        c.wait()

    def body(j, carry):
        for u in range(UNROLL):
            off = (j * UNROLL + u) * 16
            idx = biv[pl.ds(off, 16)]
            q = qv[pl.ds(off, 16)]
            x = xv[pl.ds(off, 16)]
            y = yv[pl.ds(off, 16)]
            z = zv[pl.ds(off, 16)]
            dx = x - plsc.load_gather(mcx, [idx])
            dy = y - plsc.load_gather(mcy, [idx])
            dz = z - plsc.load_gather(mcz, [idx])
            v = q * (dx * dx + dy * dy + dz * dz)
            plsc.addupdate_scatter(lacc, [lanes, idx], v)
        return carry

    lax.fori_loop(0, STEPS // UNROLL, body, 0)

    def fbody(g, carry):
        off = g * 16
        s = lacc[0, pl.ds(off, 16)]
        for l in range(1, 16):
            s = s + lacc[l, pl.ds(off, 16)]
        acc[pl.ds(off, 16)] = s
        return carry

    lax.fori_loop(0, BROWS, fbody, 0)

    pltpu.sync_copy(acc, shared.at[wid])
    plsc.subcore_barrier()

    @pl.when(wid == 0)
    def _writeout():
        pltpu.sync_copy(shared, fold)

        def gbody(g, carry):
            off = g * 16
            s = fold[0, pl.ds(off, 16)]
            for l in range(1, NWORK):
                s = s + fold[l, pl.ds(off, 16)]
            acc[pl.ds(off, 16)] = s
            return carry

        lax.fori_loop(0, BROWS, gbody, 0)
        pltpu.sync_copy(acc, out_hbm)


def kernel(pos, mass_center, scaler, vector, batch_index, W1, b1, W2, b2):
    del vector
    q = _mlp(scaler, W1, b1, W2, b2)[:, 0]

    pad = N_PAD - N
    qp = jnp.concatenate([q, jnp.zeros((pad,), jnp.float32)])
    posT = pos.T
    px = jnp.concatenate([posT[0], jnp.zeros((pad,), jnp.float32)])
    py = jnp.concatenate([posT[1], jnp.zeros((pad,), jnp.float32)])
    pz = jnp.concatenate([posT[2], jnp.zeros((pad,), jnp.float32)])
    bi = jnp.concatenate([batch_index.astype(jnp.int32),
                          jnp.zeros((pad,), jnp.int32)])
    mcT = mass_center.T

    return _sc_segsum(qp, px, py, pz, bi, mcT[0], mcT[1], mcT[2]).reshape(B, 1)
